# baseline (speedup 1.0000x reference)
import functools

import jax
import jax.numpy as jnp
from jax import lax
from jax.experimental import pallas as pl
from jax.experimental.pallas import tpu as pltpu
from jax.experimental.pallas import tpu_sc as plsc

NUM_WKR = 2000
NUM_TSK = 8000
NUM_RELS = 10
DIM = 256
E = 160000

C = 128
NCHUNK = E // C
NTILES = 32

WKR_PAD = 2048
TSK_PAD2 = 8192
DEGT_PAD = 8192


def _mm1_body(lab_ref, w_ref, t1_ref):
    t1_ref[0] = jnp.dot(lab_ref[...], w_ref[0],
                        preferred_element_type=jnp.float32)


def _tc_transform1(labels, weight_worker):
    nb = 320
    grid = (NUM_TSK // nb, NUM_RELS)
    return pl.pallas_call(
        _mm1_body,
        grid=grid,
        in_specs=[
            pl.BlockSpec((nb, DIM), lambda n, r: (n, 0)),
            pl.BlockSpec((1, DIM, DIM), lambda n, r: (r, 0, 0)),
        ],
        out_specs=pl.BlockSpec((1, nb, DIM), lambda n, r: (r, n, 0)),
        out_shape=jax.ShapeDtypeStruct((NUM_RELS, NUM_TSK, DIM), jnp.float32),
    )(labels, weight_worker)


def _mm2_body(p_ref, deg_ref, w_ref, abil_ref, t2_ref):
    s = p_ref[0] + p_ref[1]
    d = jnp.maximum(deg_ref[0] + deg_ref[1], 1.0)
    ab = s / d
    abil_ref[...] = ab
    t2_ref[0] = jnp.dot(ab, w_ref[0], preferred_element_type=jnp.float32)


def _tc_normalize_transform2(parts, degw, weight_task):
    nb = 400
    grid = (NUM_WKR // nb, NUM_RELS)
    return pl.pallas_call(
        _mm2_body,
        grid=grid,
        in_specs=[
            pl.BlockSpec((2, nb, DIM), lambda n, r: (0, n, 0)),
            pl.BlockSpec((2, nb, 1), lambda n, r: (0, n, 0)),
            pl.BlockSpec((1, DIM, DIM), lambda n, r: (r, 0, 0)),
        ],
        out_specs=[
            pl.BlockSpec((nb, DIM), lambda n, r: (n, 0)),
            pl.BlockSpec((1, nb, DIM), lambda n, r: (r, n, 0)),
        ],
        out_shape=[
            jax.ShapeDtypeStruct((NUM_WKR, DIM), jnp.float32),
            jax.ShapeDtypeStruct((NUM_RELS, NUM_WKR, DIM), jnp.float32),
        ],
    )(parts, degw, weight_task)


def _norm_body(q_ref, deg_ref, out_ref):
    d = jnp.maximum(deg_ref[0] + deg_ref[1], 1.0)
    out_ref[:, 0:128] = q_ref[0] / d
    out_ref[:, 128:256] = q_ref[1] / d


def _tc_normalize(parts, degt):
    nb = 400
    grid = (NUM_TSK // nb,)
    return pl.pallas_call(
        _norm_body,
        grid=grid,
        in_specs=[
            pl.BlockSpec((2, nb, 128), lambda n: (0, n, 0)),
            pl.BlockSpec((2, nb, 1), lambda n: (0, n, 0)),
        ],
        out_specs=pl.BlockSpec((nb, DIM), lambda n: (n, 0)),
        out_shape=jax.ShapeDtypeStruct((NUM_TSK, DIM), jnp.float32),
    )(parts, degt)


def _sc_pass1(t1_flat, esrc, edst, etyp):
    mesh = plsc.VectorSubcoreMesh(core_axis_name="c", subcore_axis_name="s")

    @functools.partial(
        pl.kernel,
        out_type=(
            jax.ShapeDtypeStruct((2, WKR_PAD, DIM), jnp.float32),
            jax.ShapeDtypeStruct((2, WKR_PAD), jnp.float32),
            jax.ShapeDtypeStruct((2, DEGT_PAD), jnp.float32),
        ),
        mesh=mesh,
        scratch_types=(
            pltpu.VMEM((C,), jnp.int32),
            pltpu.VMEM((C,), jnp.int32),
            pltpu.VMEM((C,), jnp.int32),
            pltpu.VMEM((C,), jnp.int32),
            pltpu.VMEM((C,), jnp.float32),
            pltpu.VMEM((C, 128), jnp.float32),
            pltpu.VMEM((C, 128), jnp.float32),
            pltpu.VMEM((64, 128), jnp.float32),
            pltpu.VMEM_SHARED((WKR_PAD, 128), jnp.float32),
            pltpu.VMEM_SHARED((WKR_PAD, 128), jnp.float32),
            pltpu.VMEM_SHARED((WKR_PAD,), jnp.float32),
            pltpu.VMEM_SHARED((DEGT_PAD,), jnp.float32),
            pltpu.SemaphoreType.DMA,
        ),
    )
    def k(t1_hbm, src_hbm, dst_hbm, typ_hbm,
          abil_out, degw_out, degt_out,
          srcv, dstv, typv, keyv, onesv, rowsL, rowsR, zbuf,
          accL, accR, degw_s, degt_s, sem):
        c = lax.axis_index("c")
        s = lax.axis_index("s")
        wid = c * 16 + s

        z16 = jnp.zeros((16,), jnp.float32)
        o16 = jnp.ones((16,), jnp.float32)

        def zrow(i, carry):
            for j in range(8):
                zbuf[i, pl.ds(j * 16, 16)] = z16
            return carry
        lax.fori_loop(0, 64, zrow, 0)
        for j in range(C // 16):
            onesv[pl.ds(j * 16, 16)] = o16

        for half in (accL, accR):
            pltpu.sync_copy(zbuf, half.at[pl.ds(s * 128, 64)])
            pltpu.sync_copy(zbuf, half.at[pl.ds(s * 128 + 64, 64)])
        pltpu.sync_copy(zbuf.at[0], degw_s.at[pl.ds(s * 128, 128)])
        for b in range(4):
            pltpu.sync_copy(zbuf.at[b],
                            degt_s.at[pl.ds(s * 512 + b * 128, 128)])
        plsc.subcore_barrier()

        def chunk(j, carry):
            cid = wid + j * NTILES

            @pl.when(cid < NCHUNK)
            def _():
                base = cid * C
                ld1 = pltpu.async_copy(src_hbm.at[pl.ds(base, C)], srcv, sem)
                ld2 = pltpu.async_copy(dst_hbm.at[pl.ds(base, C)], dstv, sem)
                ld3 = pltpu.async_copy(typ_hbm.at[pl.ds(base, C)], typv, sem)
                ld1.wait()
                ld2.wait()
                ld3.wait()
                for i in range(C // 16):
                    sl = pl.ds(i * 16, 16)
                    keyv[sl] = typv[sl] * NUM_TSK + srcv[sl]
                cpL = pltpu.async_copy(
                    t1_hbm.at[keyv, pl.ds(0, 128)], rowsL, sem)
                cpR = pltpu.async_copy(
                    t1_hbm.at[keyv, pl.ds(128, 128)], rowsR, sem)
                cpL.wait()
                cpR.wait()
                scL = pltpu.async_copy(rowsL, accL.at[dstv], sem, add=True)
                scR = pltpu.async_copy(rowsR, accR.at[dstv], sem, add=True)
                pltpu.sync_copy(onesv, degw_s.at[dstv], add=True)
                pltpu.sync_copy(onesv, degt_s.at[srcv], add=True)
                scL.wait()
                scR.wait()
            return carry
        lax.fori_loop(0, -(-NCHUNK // NTILES), chunk, 0)

        plsc.subcore_barrier()
        pltpu.sync_copy(accL.at[pl.ds(s * 128, 128)],
                        abil_out.at[c, pl.ds(s * 128, 128), pl.ds(0, 128)])
        pltpu.sync_copy(accR.at[pl.ds(s * 128, 128)],
                        abil_out.at[c, pl.ds(s * 128, 128), pl.ds(128, 128)])
        pltpu.sync_copy(degw_s.at[pl.ds(s * 128, 128)],
                        degw_out.at[c, pl.ds(s * 128, 128)])
        pltpu.sync_copy(degt_s.at[pl.ds(s * 512, 512)],
                        degt_out.at[c, pl.ds(s * 512, 512)])

    return k(t1_flat, esrc, edst, etyp)


def _sc_pass2(t2_flat, esrc, edst, etyp):
    mesh = plsc.VectorSubcoreMesh(core_axis_name="c", subcore_axis_name="s")

    @functools.partial(
        pl.kernel,
        out_type=jax.ShapeDtypeStruct((2, TSK_PAD2, 128), jnp.float32),
        mesh=mesh,
        scratch_types=(
            pltpu.VMEM((C,), jnp.int32),
            pltpu.VMEM((C,), jnp.int32),
            pltpu.VMEM((C,), jnp.int32),
            pltpu.VMEM((C,), jnp.int32),
            pltpu.VMEM((C, 128), jnp.float32),
            pltpu.VMEM((64, 128), jnp.float32),
            pltpu.VMEM_SHARED((TSK_PAD2, 128), jnp.float32),
            pltpu.SemaphoreType.DMA,
        ),
    )
    def k(t2_hbm, src_hbm, dst_hbm, typ_hbm, out_hbm,
          srcv, dstv, typv, keyv, rows, zbuf, acc, sem):
        c = lax.axis_index("c")
        s = lax.axis_index("s")
        wid = c * 16 + s
        colbase = pl.multiple_of(c * 128, 128)

        z16 = jnp.zeros((16,), jnp.float32)

        def zrow(i, carry):
            for j in range(8):
                zbuf[i, pl.ds(j * 16, 16)] = z16
            return carry
        lax.fori_loop(0, 64, zrow, 0)

        for b in range(8):
            pltpu.sync_copy(zbuf, acc.at[pl.ds(s * 512 + b * 64, 64)])
        plsc.subcore_barrier()

        def chunk(j, carry):
            cid = s + j * 16

            @pl.when(cid < NCHUNK)
            def _():
                base = cid * C
                ld1 = pltpu.async_copy(src_hbm.at[pl.ds(base, C)], srcv, sem)
                ld2 = pltpu.async_copy(dst_hbm.at[pl.ds(base, C)], dstv, sem)
                ld3 = pltpu.async_copy(typ_hbm.at[pl.ds(base, C)], typv, sem)
                ld1.wait()
                ld2.wait()
                ld3.wait()
                for i in range(C // 16):
                    sl = pl.ds(i * 16, 16)
                    keyv[sl] = typv[sl] * NUM_WKR + dstv[sl]
                pltpu.async_copy(
                    t2_hbm.at[keyv, pl.ds(colbase, 128)], rows, sem).wait()
                pltpu.sync_copy(rows, acc.at[srcv], add=True)
            return carry
        lax.fori_loop(0, -(-NCHUNK // 16), chunk, 0)

        plsc.subcore_barrier()
        pltpu.sync_copy(acc.at[pl.ds(s * 512, 512)],
                        out_hbm.at[c, pl.ds(s * 512, 512)])

    return k(t2_flat, esrc, edst, etyp)


def kernel(labels, edge_src_task, edge_dst_wkr, edge_type,
           weight_worker, weight_task):
    esrc = edge_src_task.astype(jnp.int32)
    edst = edge_dst_wkr.astype(jnp.int32)
    etyp = edge_type.astype(jnp.int32)

    t1 = _tc_transform1(labels, weight_worker)
    abil_p, degw_p, degt_p = _sc_pass1(
        t1.reshape(NUM_RELS * NUM_TSK, DIM), esrc, edst, etyp)

    ability, t2 = _tc_normalize_transform2(
        abil_p, degw_p.reshape(2, WKR_PAD, 1), weight_task)

    q_p = _sc_pass2(t2.reshape(NUM_RELS * NUM_WKR, DIM), esrc, edst, etyp)
    new_labels = _tc_normalize(q_p, degt_p.reshape(2, DEGT_PAD, 1))
    return ability, new_labels


# speedup vs baseline: 1.1924x; 1.1924x over previous
import functools

import jax
import jax.numpy as jnp
from jax import lax
from jax.experimental import pallas as pl
from jax.experimental.pallas import tpu as pltpu
from jax.experimental.pallas import tpu_sc as plsc

NUM_WKR = 2000
NUM_TSK = 8000
NUM_RELS = 10
DIM = 256
E = 160000

C = 128
NCHUNK = E // C
NTILES = 32

WKR_PAD = 2048
TSK_PAD2 = 8192
DEGT_PAD = 8192


def _mm1_body(lab_ref, w_ref, t1_ref):
    t1_ref[0] = jnp.dot(lab_ref[...], w_ref[0],
                        preferred_element_type=jnp.float32)


def _tc_transform1(labels, weight_worker):
    nb = 1000
    grid = (NUM_TSK // nb, NUM_RELS)
    return pl.pallas_call(
        _mm1_body,
        grid=grid,
        in_specs=[
            pl.BlockSpec((nb, DIM), lambda n, r: (n, 0)),
            pl.BlockSpec((1, DIM, DIM), lambda n, r: (r, 0, 0)),
        ],
        out_specs=pl.BlockSpec((1, nb, DIM), lambda n, r: (r, n, 0)),
        out_shape=jax.ShapeDtypeStruct((NUM_RELS, NUM_TSK, DIM), jnp.float32),
    )(labels, weight_worker)


def _mm2_body(p_ref, deg_ref, w_ref, abil_ref, t2_ref):
    s = p_ref[0] + p_ref[1]
    d = jnp.maximum(deg_ref[0] + deg_ref[1], 1.0)
    ab = s / d
    abil_ref[...] = ab
    t2_ref[0] = jnp.dot(ab, w_ref[0], preferred_element_type=jnp.float32)


def _tc_normalize_transform2(parts, degw, weight_task):
    nb = 400
    grid = (NUM_WKR // nb, NUM_RELS)
    return pl.pallas_call(
        _mm2_body,
        grid=grid,
        in_specs=[
            pl.BlockSpec((2, nb, DIM), lambda n, r: (0, n, 0)),
            pl.BlockSpec((2, nb, 1), lambda n, r: (0, n, 0)),
            pl.BlockSpec((1, DIM, DIM), lambda n, r: (r, 0, 0)),
        ],
        out_specs=[
            pl.BlockSpec((nb, DIM), lambda n, r: (n, 0)),
            pl.BlockSpec((1, nb, DIM), lambda n, r: (r, n, 0)),
        ],
        out_shape=[
            jax.ShapeDtypeStruct((NUM_WKR, DIM), jnp.float32),
            jax.ShapeDtypeStruct((NUM_RELS, NUM_WKR, DIM), jnp.float32),
        ],
    )(parts, degw, weight_task)


def _norm_body(q_ref, deg_ref, out_ref):
    d = jnp.maximum(deg_ref[0] + deg_ref[1], 1.0)
    out_ref[:, 0:128] = q_ref[0] / d
    out_ref[:, 128:256] = q_ref[1] / d


def _tc_normalize(parts, degt):
    nb = 400
    grid = (NUM_TSK // nb,)
    return pl.pallas_call(
        _norm_body,
        grid=grid,
        in_specs=[
            pl.BlockSpec((2, nb, 128), lambda n: (0, n, 0)),
            pl.BlockSpec((2, nb, 1), lambda n: (0, n, 0)),
        ],
        out_specs=pl.BlockSpec((nb, DIM), lambda n: (n, 0)),
        out_shape=jax.ShapeDtypeStruct((NUM_TSK, DIM), jnp.float32),
    )(parts, degt)


def _sc_pass1(t1_flat, esrc, edst, etyp):
    mesh = plsc.VectorSubcoreMesh(core_axis_name="c", subcore_axis_name="s")

    @functools.partial(
        pl.kernel,
        out_type=(
            jax.ShapeDtypeStruct((2, WKR_PAD, DIM), jnp.float32),
            jax.ShapeDtypeStruct((2, WKR_PAD), jnp.float32),
            jax.ShapeDtypeStruct((2, DEGT_PAD), jnp.float32),
        ),
        mesh=mesh,
        scratch_types=(
            pltpu.VMEM((C,), jnp.int32),
            pltpu.VMEM((C,), jnp.int32),
            pltpu.VMEM((C,), jnp.int32),
            pltpu.VMEM((C,), jnp.int32),
            pltpu.VMEM((C,), jnp.float32),
            pltpu.VMEM((C, 128), jnp.float32),
            pltpu.VMEM((C, 128), jnp.float32),
            pltpu.VMEM((64, 128), jnp.float32),
            pltpu.VMEM_SHARED((WKR_PAD, 128), jnp.float32),
            pltpu.VMEM_SHARED((WKR_PAD, 128), jnp.float32),
            pltpu.VMEM_SHARED((WKR_PAD,), jnp.float32),
            pltpu.VMEM_SHARED((DEGT_PAD,), jnp.float32),
            pltpu.SemaphoreType.DMA,
        ),
    )
    def k(t1_hbm, src_hbm, dst_hbm, typ_hbm,
          abil_out, degw_out, degt_out,
          srcv, dstv, typv, keyv, onesv, rowsL, rowsR, zbuf,
          accL, accR, degw_s, degt_s, sem):
        c = lax.axis_index("c")
        s = lax.axis_index("s")
        wid = c * 16 + s

        z16 = jnp.zeros((16,), jnp.float32)
        o16 = jnp.ones((16,), jnp.float32)

        def zrow(i, carry):
            for j in range(8):
                zbuf[i, pl.ds(j * 16, 16)] = z16
            return carry
        lax.fori_loop(0, 64, zrow, 0)
        for j in range(C // 16):
            onesv[pl.ds(j * 16, 16)] = o16

        for half in (accL, accR):
            pltpu.sync_copy(zbuf, half.at[pl.ds(s * 128, 64)])
            pltpu.sync_copy(zbuf, half.at[pl.ds(s * 128 + 64, 64)])
        pltpu.sync_copy(zbuf.at[0], degw_s.at[pl.ds(s * 128, 128)])
        for b in range(4):
            pltpu.sync_copy(zbuf.at[b],
                            degt_s.at[pl.ds(s * 512 + b * 128, 128)])
        plsc.subcore_barrier()

        def chunk(j, carry):
            cid = wid + j * NTILES

            @pl.when(cid < NCHUNK)
            def _():
                base = cid * C
                ld1 = pltpu.async_copy(src_hbm.at[pl.ds(base, C)], srcv, sem)
                ld2 = pltpu.async_copy(dst_hbm.at[pl.ds(base, C)], dstv, sem)
                ld3 = pltpu.async_copy(typ_hbm.at[pl.ds(base, C)], typv, sem)
                ld1.wait()
                ld2.wait()
                ld3.wait()
                for i in range(C // 16):
                    sl = pl.ds(i * 16, 16)
                    keyv[sl] = typv[sl] * NUM_TSK + srcv[sl]
                cpL = pltpu.async_copy(
                    t1_hbm.at[keyv, pl.ds(0, 128)], rowsL, sem)
                cpR = pltpu.async_copy(
                    t1_hbm.at[keyv, pl.ds(128, 128)], rowsR, sem)
                cpL.wait()
                cpR.wait()
                scL = pltpu.async_copy(rowsL, accL.at[dstv], sem, add=True)
                scR = pltpu.async_copy(rowsR, accR.at[dstv], sem, add=True)
                pltpu.sync_copy(onesv, degw_s.at[dstv], add=True)
                pltpu.sync_copy(onesv, degt_s.at[srcv], add=True)
                scL.wait()
                scR.wait()
            return carry
        lax.fori_loop(0, -(-NCHUNK // NTILES), chunk, 0)

        plsc.subcore_barrier()
        pltpu.sync_copy(accL.at[pl.ds(s * 128, 128)],
                        abil_out.at[c, pl.ds(s * 128, 128), pl.ds(0, 128)])
        pltpu.sync_copy(accR.at[pl.ds(s * 128, 128)],
                        abil_out.at[c, pl.ds(s * 128, 128), pl.ds(128, 128)])
        pltpu.sync_copy(degw_s.at[pl.ds(s * 128, 128)],
                        degw_out.at[c, pl.ds(s * 128, 128)])
        pltpu.sync_copy(degt_s.at[pl.ds(s * 512, 512)],
                        degt_out.at[c, pl.ds(s * 512, 512)])

    return k(t1_flat, esrc, edst, etyp)


def _sc_pass2(t2_flat, esrc, edst, etyp):
    mesh = plsc.VectorSubcoreMesh(core_axis_name="c", subcore_axis_name="s")

    @functools.partial(
        pl.kernel,
        out_type=jax.ShapeDtypeStruct((2, TSK_PAD2, 128), jnp.float32),
        mesh=mesh,
        scratch_types=(
            pltpu.VMEM((C,), jnp.int32),
            pltpu.VMEM((C,), jnp.int32),
            pltpu.VMEM((C,), jnp.int32),
            pltpu.VMEM((C,), jnp.int32),
            pltpu.VMEM((C, 128), jnp.float32),
            pltpu.VMEM((64, 128), jnp.float32),
            pltpu.VMEM_SHARED((TSK_PAD2, 128), jnp.float32),
            pltpu.SemaphoreType.DMA,
        ),
    )
    def k(t2_hbm, src_hbm, dst_hbm, typ_hbm, out_hbm,
          srcv, dstv, typv, keyv, rows, zbuf, acc, sem):
        c = lax.axis_index("c")
        s = lax.axis_index("s")
        wid = c * 16 + s
        colbase = pl.multiple_of(c * 128, 128)

        z16 = jnp.zeros((16,), jnp.float32)

        def zrow(i, carry):
            for j in range(8):
                zbuf[i, pl.ds(j * 16, 16)] = z16
            return carry
        lax.fori_loop(0, 64, zrow, 0)

        for b in range(8):
            pltpu.sync_copy(zbuf, acc.at[pl.ds(s * 512 + b * 64, 64)])
        plsc.subcore_barrier()

        def chunk(j, carry):
            cid = s + j * 16

            @pl.when(cid < NCHUNK)
            def _():
                base = cid * C
                ld1 = pltpu.async_copy(src_hbm.at[pl.ds(base, C)], srcv, sem)
                ld2 = pltpu.async_copy(dst_hbm.at[pl.ds(base, C)], dstv, sem)
                ld3 = pltpu.async_copy(typ_hbm.at[pl.ds(base, C)], typv, sem)
                ld1.wait()
                ld2.wait()
                ld3.wait()
                for i in range(C // 16):
                    sl = pl.ds(i * 16, 16)
                    keyv[sl] = typv[sl] * NUM_WKR + dstv[sl]
                pltpu.async_copy(
                    t2_hbm.at[keyv, pl.ds(colbase, 128)], rows, sem).wait()
                pltpu.sync_copy(rows, acc.at[srcv], add=True)
            return carry
        lax.fori_loop(0, -(-NCHUNK // 16), chunk, 0)

        plsc.subcore_barrier()
        pltpu.sync_copy(acc.at[pl.ds(s * 512, 512)],
                        out_hbm.at[c, pl.ds(s * 512, 512)])

    return k(t2_flat, esrc, edst, etyp)


def kernel(labels, edge_src_task, edge_dst_wkr, edge_type,
           weight_worker, weight_task):
    esrc = edge_src_task.astype(jnp.int32)
    edst = edge_dst_wkr.astype(jnp.int32)
    etyp = edge_type.astype(jnp.int32)

    t1 = _tc_transform1(labels, weight_worker)
    abil_p, degw_p, degt_p = _sc_pass1(
        t1.reshape(NUM_RELS * NUM_TSK, DIM), esrc, edst, etyp)

    ability, t2 = _tc_normalize_transform2(
        abil_p, degw_p.reshape(2, WKR_PAD, 1), weight_task)

    q_p = _sc_pass2(t2.reshape(NUM_RELS * NUM_WKR, DIM), esrc, edst, etyp)
    new_labels = _tc_normalize(q_p, degt_p.reshape(2, DEGT_PAD, 1))
    return ability, new_labels


# speedup vs baseline: 1.2492x; 1.0477x over previous
import functools

import jax
import jax.numpy as jnp
from jax import lax
from jax.experimental import pallas as pl
from jax.experimental.pallas import tpu as pltpu
from jax.experimental.pallas import tpu_sc as plsc

NUM_WKR = 2000
NUM_TSK = 8000
NUM_RELS = 10
DIM = 256
E = 160000

C = 128
NCHUNK = E // C
NTILES = 32

WKR_PAD = 2048
TSK_PAD2 = 8192
DEGT_PAD = 8192


def _mm1_body(lab_ref, w_ref, t1_ref):
    t1_ref[0] = jnp.dot(lab_ref[...], w_ref[0],
                        preferred_element_type=jnp.float32)


def _tc_transform1(labels, weight_worker):
    nb = 1000
    grid = (NUM_TSK // nb, NUM_RELS)
    return pl.pallas_call(
        _mm1_body,
        grid=grid,
        in_specs=[
            pl.BlockSpec((nb, DIM), lambda n, r: (n, 0)),
            pl.BlockSpec((1, DIM, DIM), lambda n, r: (r, 0, 0)),
        ],
        out_specs=pl.BlockSpec((1, nb, DIM), lambda n, r: (r, n, 0)),
        out_shape=jax.ShapeDtypeStruct((NUM_RELS, NUM_TSK, DIM), jnp.float32),
    )(labels, weight_worker)


def _mm2_body(p_ref, deg_ref, w_ref, abil_ref, t2_ref):
    s = p_ref[0] + p_ref[1]
    d = jnp.maximum(deg_ref[0] + deg_ref[1], 1.0)
    ab = s / d
    abil_ref[...] = ab
    t2_ref[0] = jnp.dot(ab, w_ref[0], preferred_element_type=jnp.float32)


def _tc_normalize_transform2(parts, degw, weight_task):
    nb = 1000
    grid = (NUM_WKR // nb, NUM_RELS)
    return pl.pallas_call(
        _mm2_body,
        grid=grid,
        in_specs=[
            pl.BlockSpec((2, nb, DIM), lambda n, r: (0, n, 0)),
            pl.BlockSpec((2, nb, 1), lambda n, r: (0, n, 0)),
            pl.BlockSpec((1, DIM, DIM), lambda n, r: (r, 0, 0)),
        ],
        out_specs=[
            pl.BlockSpec((nb, DIM), lambda n, r: (n, 0)),
            pl.BlockSpec((1, nb, DIM), lambda n, r: (r, n, 0)),
        ],
        out_shape=[
            jax.ShapeDtypeStruct((NUM_WKR, DIM), jnp.float32),
            jax.ShapeDtypeStruct((NUM_RELS, NUM_WKR, DIM), jnp.float32),
        ],
    )(parts, degw, weight_task)


def _norm_body(q_ref, deg_ref, out_ref):
    d = jnp.maximum(deg_ref[0] + deg_ref[1], 1.0)
    out_ref[:, 0:128] = q_ref[0] / d
    out_ref[:, 128:256] = q_ref[1] / d


def _tc_normalize(parts, degt):
    nb = 1000
    grid = (NUM_TSK // nb,)
    return pl.pallas_call(
        _norm_body,
        grid=grid,
        in_specs=[
            pl.BlockSpec((2, nb, 128), lambda n: (0, n, 0)),
            pl.BlockSpec((2, nb, 1), lambda n: (0, n, 0)),
        ],
        out_specs=pl.BlockSpec((nb, DIM), lambda n: (n, 0)),
        out_shape=jax.ShapeDtypeStruct((NUM_TSK, DIM), jnp.float32),
    )(parts, degt)


def _sc_pass1(t1_flat, esrc, edst, etyp):
    mesh = plsc.VectorSubcoreMesh(core_axis_name="c", subcore_axis_name="s")

    @functools.partial(
        pl.kernel,
        out_type=(
            jax.ShapeDtypeStruct((2, WKR_PAD, DIM), jnp.float32),
            jax.ShapeDtypeStruct((2, WKR_PAD), jnp.float32),
            jax.ShapeDtypeStruct((2, DEGT_PAD), jnp.float32),
        ),
        mesh=mesh,
        scratch_types=(
            pltpu.VMEM((C,), jnp.int32),
            pltpu.VMEM((C,), jnp.int32),
            pltpu.VMEM((C,), jnp.int32),
            pltpu.VMEM((C,), jnp.int32),
            pltpu.VMEM((C,), jnp.float32),
            pltpu.VMEM((C, 128), jnp.float32),
            pltpu.VMEM((C, 128), jnp.float32),
            pltpu.VMEM((64, 128), jnp.float32),
            pltpu.VMEM_SHARED((WKR_PAD, 128), jnp.float32),
            pltpu.VMEM_SHARED((WKR_PAD, 128), jnp.float32),
            pltpu.VMEM_SHARED((WKR_PAD,), jnp.float32),
            pltpu.VMEM_SHARED((DEGT_PAD,), jnp.float32),
            pltpu.SemaphoreType.DMA,
        ),
    )
    def k(t1_hbm, src_hbm, dst_hbm, typ_hbm,
          abil_out, degw_out, degt_out,
          srcv, dstv, typv, keyv, onesv, rowsL, rowsR, zbuf,
          accL, accR, degw_s, degt_s, sem):
        c = lax.axis_index("c")
        s = lax.axis_index("s")
        wid = c * 16 + s

        z16 = jnp.zeros((16,), jnp.float32)
        o16 = jnp.ones((16,), jnp.float32)

        def zrow(i, carry):
            for j in range(8):
                zbuf[i, pl.ds(j * 16, 16)] = z16
            return carry
        lax.fori_loop(0, 64, zrow, 0)
        for j in range(C // 16):
            onesv[pl.ds(j * 16, 16)] = o16

        for half in (accL, accR):
            pltpu.sync_copy(zbuf, half.at[pl.ds(s * 128, 64)])
            pltpu.sync_copy(zbuf, half.at[pl.ds(s * 128 + 64, 64)])
        pltpu.sync_copy(zbuf.at[0], degw_s.at[pl.ds(s * 128, 128)])
        for b in range(4):
            pltpu.sync_copy(zbuf.at[b],
                            degt_s.at[pl.ds(s * 512 + b * 128, 128)])
        plsc.subcore_barrier()

        def chunk(j, carry):
            cid = wid + j * NTILES

            @pl.when(cid < NCHUNK)
            def _():
                base = cid * C
                ld1 = pltpu.async_copy(src_hbm.at[pl.ds(base, C)], srcv, sem)
                ld2 = pltpu.async_copy(dst_hbm.at[pl.ds(base, C)], dstv, sem)
                ld3 = pltpu.async_copy(typ_hbm.at[pl.ds(base, C)], typv, sem)
                ld1.wait()
                ld2.wait()
                ld3.wait()
                for i in range(C // 16):
                    sl = pl.ds(i * 16, 16)
                    keyv[sl] = typv[sl] * NUM_TSK + srcv[sl]
                cpL = pltpu.async_copy(
                    t1_hbm.at[keyv, pl.ds(0, 128)], rowsL, sem)
                cpR = pltpu.async_copy(
                    t1_hbm.at[keyv, pl.ds(128, 128)], rowsR, sem)
                cpL.wait()
                cpR.wait()
                scL = pltpu.async_copy(rowsL, accL.at[dstv], sem, add=True)
                scR = pltpu.async_copy(rowsR, accR.at[dstv], sem, add=True)
                pltpu.sync_copy(onesv, degw_s.at[dstv], add=True)
                pltpu.sync_copy(onesv, degt_s.at[srcv], add=True)
                scL.wait()
                scR.wait()
            return carry
        lax.fori_loop(0, -(-NCHUNK // NTILES), chunk, 0)

        plsc.subcore_barrier()
        pltpu.sync_copy(accL.at[pl.ds(s * 128, 128)],
                        abil_out.at[c, pl.ds(s * 128, 128), pl.ds(0, 128)])
        pltpu.sync_copy(accR.at[pl.ds(s * 128, 128)],
                        abil_out.at[c, pl.ds(s * 128, 128), pl.ds(128, 128)])
        pltpu.sync_copy(degw_s.at[pl.ds(s * 128, 128)],
                        degw_out.at[c, pl.ds(s * 128, 128)])
        pltpu.sync_copy(degt_s.at[pl.ds(s * 512, 512)],
                        degt_out.at[c, pl.ds(s * 512, 512)])

    return k(t1_flat, esrc, edst, etyp)


def _sc_pass2(t2_flat, esrc, edst, etyp):
    mesh = plsc.VectorSubcoreMesh(core_axis_name="c", subcore_axis_name="s")

    @functools.partial(
        pl.kernel,
        out_type=jax.ShapeDtypeStruct((2, TSK_PAD2, 128), jnp.float32),
        mesh=mesh,
        scratch_types=(
            pltpu.VMEM((C,), jnp.int32),
            pltpu.VMEM((C,), jnp.int32),
            pltpu.VMEM((C,), jnp.int32),
            pltpu.VMEM((C,), jnp.int32),
            pltpu.VMEM((C, 128), jnp.float32),
            pltpu.VMEM((64, 128), jnp.float32),
            pltpu.VMEM_SHARED((TSK_PAD2, 128), jnp.float32),
            pltpu.SemaphoreType.DMA,
        ),
    )
    def k(t2_hbm, src_hbm, dst_hbm, typ_hbm, out_hbm,
          srcv, dstv, typv, keyv, rows, zbuf, acc, sem):
        c = lax.axis_index("c")
        s = lax.axis_index("s")
        wid = c * 16 + s
        colbase = pl.multiple_of(c * 128, 128)

        z16 = jnp.zeros((16,), jnp.float32)

        def zrow(i, carry):
            for j in range(8):
                zbuf[i, pl.ds(j * 16, 16)] = z16
            return carry
        lax.fori_loop(0, 64, zrow, 0)

        for b in range(8):
            pltpu.sync_copy(zbuf, acc.at[pl.ds(s * 512 + b * 64, 64)])
        plsc.subcore_barrier()

        def chunk(j, carry):
            cid = s + j * 16

            @pl.when(cid < NCHUNK)
            def _():
                base = cid * C
                ld1 = pltpu.async_copy(src_hbm.at[pl.ds(base, C)], srcv, sem)
                ld2 = pltpu.async_copy(dst_hbm.at[pl.ds(base, C)], dstv, sem)
                ld3 = pltpu.async_copy(typ_hbm.at[pl.ds(base, C)], typv, sem)
                ld1.wait()
                ld2.wait()
                ld3.wait()
                for i in range(C // 16):
                    sl = pl.ds(i * 16, 16)
                    keyv[sl] = typv[sl] * NUM_WKR + dstv[sl]
                pltpu.async_copy(
                    t2_hbm.at[keyv, pl.ds(colbase, 128)], rows, sem).wait()
                pltpu.sync_copy(rows, acc.at[srcv], add=True)
            return carry
        lax.fori_loop(0, -(-NCHUNK // 16), chunk, 0)

        plsc.subcore_barrier()
        pltpu.sync_copy(acc.at[pl.ds(s * 512, 512)],
                        out_hbm.at[c, pl.ds(s * 512, 512)])

    return k(t2_flat, esrc, edst, etyp)


def kernel(labels, edge_src_task, edge_dst_wkr, edge_type,
           weight_worker, weight_task):
    esrc = edge_src_task.astype(jnp.int32)
    edst = edge_dst_wkr.astype(jnp.int32)
    etyp = edge_type.astype(jnp.int32)

    t1 = _tc_transform1(labels, weight_worker)
    abil_p, degw_p, degt_p = _sc_pass1(
        t1.reshape(NUM_RELS * NUM_TSK, DIM), esrc, edst, etyp)

    ability, t2 = _tc_normalize_transform2(
        abil_p, degw_p.reshape(2, WKR_PAD, 1), weight_task)

    q_p = _sc_pass2(t2.reshape(NUM_RELS * NUM_WKR, DIM), esrc, edst, etyp)
    new_labels = _tc_normalize(q_p, degt_p.reshape(2, DEGT_PAD, 1))
    return ability, new_labels


# speedup vs baseline: 1.3621x; 1.0904x over previous
import functools

import jax
import jax.numpy as jnp
from jax import lax
from jax.experimental import pallas as pl
from jax.experimental.pallas import tpu as pltpu
from jax.experimental.pallas import tpu_sc as plsc

NUM_WKR = 2000
NUM_TSK = 8000
NUM_RELS = 10
DIM = 256
E = 160000

C = 128
NCHUNK = E // C
NTILES = 32

WKR_PAD = 2048
TSK_PAD2 = 8192
DEGT_PAD = 8192


def _mm1_body(lab_ref, w_ref, t1_ref):
    t1_ref[0] = jnp.dot(lab_ref[...], w_ref[0],
                        preferred_element_type=jnp.float32)


def _tc_transform1(labels, weight_worker):
    nb = 1000
    grid = (NUM_TSK // nb, NUM_RELS)
    return pl.pallas_call(
        _mm1_body,
        grid=grid,
        in_specs=[
            pl.BlockSpec((nb, DIM), lambda n, r: (n, 0)),
            pl.BlockSpec((1, DIM, DIM), lambda n, r: (r, 0, 0)),
        ],
        out_specs=pl.BlockSpec((1, nb, DIM), lambda n, r: (r, n, 0)),
        out_shape=jax.ShapeDtypeStruct((NUM_RELS, NUM_TSK, DIM), jnp.float32),
    )(labels, weight_worker)


def _mm2_body(p_ref, deg_ref, w_ref, abil_ref, t2_ref):
    s = p_ref[0] + p_ref[1]
    d = jnp.maximum(deg_ref[0] + deg_ref[1], 1.0)
    ab = s / d
    abil_ref[...] = ab
    t2_ref[0] = jnp.dot(ab, w_ref[0], preferred_element_type=jnp.float32)


def _tc_normalize_transform2(parts, degw, weight_task):
    nb = 1000
    grid = (NUM_WKR // nb, NUM_RELS)
    return pl.pallas_call(
        _mm2_body,
        grid=grid,
        in_specs=[
            pl.BlockSpec((2, nb, DIM), lambda n, r: (0, n, 0)),
            pl.BlockSpec((2, nb, 1), lambda n, r: (0, n, 0)),
            pl.BlockSpec((1, DIM, DIM), lambda n, r: (r, 0, 0)),
        ],
        out_specs=[
            pl.BlockSpec((nb, DIM), lambda n, r: (n, 0)),
            pl.BlockSpec((1, nb, DIM), lambda n, r: (r, n, 0)),
        ],
        out_shape=[
            jax.ShapeDtypeStruct((NUM_WKR, DIM), jnp.float32),
            jax.ShapeDtypeStruct((NUM_RELS, NUM_WKR, DIM), jnp.float32),
        ],
    )(parts, degw, weight_task)


def _norm_body(q_ref, deg_ref, out_ref):
    d = jnp.maximum(deg_ref[0] + deg_ref[1], 1.0)
    out_ref[:, 0:128] = q_ref[0] / d
    out_ref[:, 128:256] = q_ref[1] / d


def _tc_normalize(parts, degt):
    nb = 1000
    grid = (NUM_TSK // nb,)
    return pl.pallas_call(
        _norm_body,
        grid=grid,
        in_specs=[
            pl.BlockSpec((2, nb, 128), lambda n: (0, n, 0)),
            pl.BlockSpec((2, nb, 1), lambda n: (0, n, 0)),
        ],
        out_specs=pl.BlockSpec((nb, DIM), lambda n: (n, 0)),
        out_shape=jax.ShapeDtypeStruct((NUM_TSK, DIM), jnp.float32),
    )(parts, degt)


def _sc_pass1(t1_flat, esrc, edst, etyp):
    mesh = plsc.VectorSubcoreMesh(core_axis_name="c", subcore_axis_name="s")

    @functools.partial(
        pl.kernel,
        out_type=(
            jax.ShapeDtypeStruct((2, WKR_PAD, DIM), jnp.float32),
            jax.ShapeDtypeStruct((2, WKR_PAD), jnp.float32),
            jax.ShapeDtypeStruct((2, DEGT_PAD), jnp.float32),
        ),
        mesh=mesh,
        scratch_types=(
            pltpu.VMEM((C,), jnp.int32),
            pltpu.VMEM((C,), jnp.int32),
            pltpu.VMEM((C,), jnp.int32),
            pltpu.VMEM((C,), jnp.int32),
            pltpu.VMEM((C,), jnp.float32),
            pltpu.VMEM((C, 128), jnp.float32),
            pltpu.VMEM((C, 128), jnp.float32),
            pltpu.VMEM((64, 128), jnp.float32),
            pltpu.VMEM_SHARED((WKR_PAD, 128), jnp.float32),
            pltpu.VMEM_SHARED((WKR_PAD, 128), jnp.float32),
            pltpu.VMEM_SHARED((WKR_PAD,), jnp.float32),
            pltpu.VMEM_SHARED((DEGT_PAD,), jnp.float32),
            pltpu.SemaphoreType.DMA,
        ),
    )
    def k(t1_hbm, src_hbm, dst_hbm, typ_hbm,
          abil_out, degw_out, degt_out,
          srcv, dstv, typv, keyv, onesv, rowsL, rowsR, zbuf,
          accL, accR, degw_s, degt_s, sem):
        c = lax.axis_index("c")
        s = lax.axis_index("s")
        wid = c * 16 + s

        z16 = jnp.zeros((16,), jnp.float32)
        o16 = jnp.ones((16,), jnp.float32)

        def zrow(i, carry):
            for j in range(8):
                zbuf[i, pl.ds(j * 16, 16)] = z16
            return carry
        lax.fori_loop(0, 64, zrow, 0)
        for j in range(C // 16):
            onesv[pl.ds(j * 16, 16)] = o16

        for half in (accL, accR):
            pltpu.sync_copy(zbuf, half.at[pl.ds(s * 128, 64)])
            pltpu.sync_copy(zbuf, half.at[pl.ds(s * 128 + 64, 64)])
        pltpu.sync_copy(zbuf.at[0], degw_s.at[pl.ds(s * 128, 128)])
        for b in range(4):
            pltpu.sync_copy(zbuf.at[b],
                            degt_s.at[pl.ds(s * 512 + b * 128, 128)])
        plsc.subcore_barrier()

        def chunk(j, carry):
            cid = wid + j * NTILES

            @pl.when(cid < NCHUNK)
            def _():
                base = cid * C
                ld1 = pltpu.async_copy(src_hbm.at[pl.ds(base, C)], srcv, sem)
                ld2 = pltpu.async_copy(dst_hbm.at[pl.ds(base, C)], dstv, sem)
                ld3 = pltpu.async_copy(typ_hbm.at[pl.ds(base, C)], typv, sem)
                ld1.wait()
                ld2.wait()
                ld3.wait()
                for i in range(C // 16):
                    sl = pl.ds(i * 16, 16)
                    keyv[sl] = typv[sl] * NUM_TSK + srcv[sl]
                cpL = pltpu.async_copy(
                    t1_hbm.at[keyv, pl.ds(0, 128)], rowsL, sem)
                cpR = pltpu.async_copy(
                    t1_hbm.at[keyv, pl.ds(128, 128)], rowsR, sem)
                cpL.wait()
                cpR.wait()
                scL = pltpu.async_copy(rowsL, accL.at[dstv], sem, add=True)
                scR = pltpu.async_copy(rowsR, accR.at[dstv], sem, add=True)
                pltpu.sync_copy(onesv, degw_s.at[dstv], add=True)
                pltpu.sync_copy(onesv, degt_s.at[srcv], add=True)
                scL.wait()
                scR.wait()
            return carry
        lax.fori_loop(0, -(-NCHUNK // NTILES), chunk, 0)

        plsc.subcore_barrier()
        pltpu.sync_copy(accL.at[pl.ds(s * 128, 128)],
                        abil_out.at[c, pl.ds(s * 128, 128), pl.ds(0, 128)])
        pltpu.sync_copy(accR.at[pl.ds(s * 128, 128)],
                        abil_out.at[c, pl.ds(s * 128, 128), pl.ds(128, 128)])
        pltpu.sync_copy(degw_s.at[pl.ds(s * 128, 128)],
                        degw_out.at[c, pl.ds(s * 128, 128)])
        pltpu.sync_copy(degt_s.at[pl.ds(s * 512, 512)],
                        degt_out.at[c, pl.ds(s * 512, 512)])

    return k(t1_flat, esrc, edst, etyp)


def _sc_pass2(t2_flat, esrc, edst, etyp):
    mesh = plsc.VectorSubcoreMesh(core_axis_name="c", subcore_axis_name="s")

    C2 = 256
    NCHUNK2 = E // C2

    @functools.partial(
        pl.kernel,
        out_type=jax.ShapeDtypeStruct((2, TSK_PAD2, 128), jnp.float32),
        mesh=mesh,
        scratch_types=(
            pltpu.VMEM((C2,), jnp.int32),
            pltpu.VMEM((C2,), jnp.int32),
            pltpu.VMEM((C2,), jnp.int32),
            pltpu.VMEM((128,), jnp.int32),
            pltpu.VMEM((128,), jnp.int32),
            pltpu.VMEM((128,), jnp.int32),
            pltpu.VMEM((128,), jnp.int32),
            pltpu.VMEM((128, 128), jnp.float32),
            pltpu.VMEM((128, 128), jnp.float32),
            pltpu.VMEM((64, 128), jnp.float32),
            pltpu.VMEM_SHARED((TSK_PAD2, 128), jnp.float32),
            pltpu.SemaphoreType.DMA,
        ),
    )
    def k(t2_hbm, src_hbm, dst_hbm, typ_hbm, out_hbm,
          srcv, dstv, typv, keyA, keyB, srcA, srcB,
          rowsA, rowsB, zbuf, acc, sem):
        c = lax.axis_index("c")
        s = lax.axis_index("s")
        colbase = pl.multiple_of(c * 128, 128)

        z16 = jnp.zeros((16,), jnp.float32)

        def zrow(i, carry):
            for j in range(8):
                zbuf[i, pl.ds(j * 16, 16)] = z16
            return carry
        lax.fori_loop(0, 64, zrow, 0)

        for b in range(8):
            pltpu.sync_copy(zbuf, acc.at[pl.ds(s * 512 + b * 64, 64)])
        plsc.subcore_barrier()

        def chunk(j, carry):
            cid = s + j * 16

            @pl.when(cid < NCHUNK2)
            def _():
                base = cid * C2
                ld1 = pltpu.async_copy(src_hbm.at[pl.ds(base, C2)], srcv, sem)
                ld2 = pltpu.async_copy(dst_hbm.at[pl.ds(base, C2)], dstv, sem)
                ld3 = pltpu.async_copy(typ_hbm.at[pl.ds(base, C2)], typv, sem)
                ld1.wait()
                ld2.wait()
                ld3.wait()
                for i in range(8):
                    sl = pl.ds(i * 16, 16)
                    sh = pl.ds(128 + i * 16, 16)
                    keyA[sl] = typv[sl] * NUM_WKR + dstv[sl]
                    keyB[sl] = typv[sh] * NUM_WKR + dstv[sh]
                    srcA[sl] = srcv[sl]
                    srcB[sl] = srcv[sh]
                cpA = pltpu.async_copy(
                    t2_hbm.at[keyA, pl.ds(colbase, 128)], rowsA, sem)
                cpB = pltpu.async_copy(
                    t2_hbm.at[keyB, pl.ds(colbase, 128)], rowsB, sem)
                cpA.wait()
                cpB.wait()
                scA = pltpu.async_copy(rowsA, acc.at[srcA], sem, add=True)
                scB = pltpu.async_copy(rowsB, acc.at[srcB], sem, add=True)
                scA.wait()
                scB.wait()
            return carry
        lax.fori_loop(0, -(-NCHUNK2 // 16), chunk, 0)

        plsc.subcore_barrier()
        pltpu.sync_copy(acc.at[pl.ds(s * 512, 512)],
                        out_hbm.at[c, pl.ds(s * 512, 512)])

    return k(t2_flat, esrc, edst, etyp)


def kernel(labels, edge_src_task, edge_dst_wkr, edge_type,
           weight_worker, weight_task):
    esrc = edge_src_task.astype(jnp.int32)
    edst = edge_dst_wkr.astype(jnp.int32)
    etyp = edge_type.astype(jnp.int32)

    t1 = _tc_transform1(labels, weight_worker)
    abil_p, degw_p, degt_p = _sc_pass1(
        t1.reshape(NUM_RELS * NUM_TSK, DIM), esrc, edst, etyp)

    ability, t2 = _tc_normalize_transform2(
        abil_p, degw_p.reshape(2, WKR_PAD, 1), weight_task)

    q_p = _sc_pass2(t2.reshape(NUM_RELS * NUM_WKR, DIM), esrc, edst, etyp)
    new_labels = _tc_normalize(q_p, degt_p.reshape(2, DEGT_PAD, 1))
    return ability, new_labels


# speedup vs baseline: 1.4018x; 1.0291x over previous
import functools

import jax
import jax.numpy as jnp
from jax import lax
from jax.experimental import pallas as pl
from jax.experimental.pallas import tpu as pltpu
from jax.experimental.pallas import tpu_sc as plsc

NUM_WKR = 2000
NUM_TSK = 8000
NUM_RELS = 10
DIM = 256
E = 160000

C = 128
NCHUNK = E // C
NTILES = 32

WKR_PAD = 2048
TSK_PAD2 = 8192
DEGT_PAD = 8192


def _mm1_body(lab_ref, w_ref, t1_ref):
    t1_ref[0] = jnp.dot(lab_ref[...], w_ref[0],
                        preferred_element_type=jnp.float32)


def _tc_transform1(labels, weight_worker):
    nb = 1000
    grid = (NUM_TSK // nb, NUM_RELS)
    return pl.pallas_call(
        _mm1_body,
        grid=grid,
        in_specs=[
            pl.BlockSpec((nb, DIM), lambda n, r: (n, 0)),
            pl.BlockSpec((1, DIM, DIM), lambda n, r: (r, 0, 0)),
        ],
        out_specs=pl.BlockSpec((1, nb, DIM), lambda n, r: (r, n, 0)),
        out_shape=jax.ShapeDtypeStruct((NUM_RELS, NUM_TSK, DIM), jnp.float32),
    )(labels, weight_worker)


def _mm2_body(p_ref, deg_ref, w_ref, abil_ref, t2_ref):
    s = p_ref[0] + p_ref[1]
    d = jnp.maximum(deg_ref[0] + deg_ref[1], 1.0)
    ab = s / d
    abil_ref[...] = ab
    t2_ref[0] = jnp.dot(ab, w_ref[0], preferred_element_type=jnp.float32)


def _tc_normalize_transform2(parts, degw, weight_task):
    nb = 1000
    grid = (NUM_WKR // nb, NUM_RELS)
    return pl.pallas_call(
        _mm2_body,
        grid=grid,
        in_specs=[
            pl.BlockSpec((2, nb, DIM), lambda n, r: (0, n, 0)),
            pl.BlockSpec((2, nb, 1), lambda n, r: (0, n, 0)),
            pl.BlockSpec((1, DIM, DIM), lambda n, r: (r, 0, 0)),
        ],
        out_specs=[
            pl.BlockSpec((nb, DIM), lambda n, r: (n, 0)),
            pl.BlockSpec((1, nb, DIM), lambda n, r: (r, n, 0)),
        ],
        out_shape=[
            jax.ShapeDtypeStruct((NUM_WKR, DIM), jnp.float32),
            jax.ShapeDtypeStruct((NUM_RELS, NUM_WKR, DIM), jnp.float32),
        ],
    )(parts, degw, weight_task)


def _norm_body(q_ref, deg_ref, out_ref):
    d = jnp.maximum(deg_ref[0] + deg_ref[1], 1.0)
    out_ref[:, 0:128] = q_ref[0] / d
    out_ref[:, 128:256] = q_ref[1] / d


def _tc_normalize(parts, degt):
    nb = 1000
    grid = (NUM_TSK // nb,)
    return pl.pallas_call(
        _norm_body,
        grid=grid,
        in_specs=[
            pl.BlockSpec((2, nb, 128), lambda n: (0, n, 0)),
            pl.BlockSpec((2, nb, 1), lambda n: (0, n, 0)),
        ],
        out_specs=pl.BlockSpec((nb, DIM), lambda n: (n, 0)),
        out_shape=jax.ShapeDtypeStruct((NUM_TSK, DIM), jnp.float32),
    )(parts, degt)


def _sc_pass1(t1_flat, esrc, edst, etyp):
    mesh = plsc.VectorSubcoreMesh(core_axis_name="c", subcore_axis_name="s")
    C2 = 256
    NCHUNK2 = E // C2

    @functools.partial(
        pl.kernel,
        out_type=(
            jax.ShapeDtypeStruct((2, WKR_PAD, DIM), jnp.float32),
            jax.ShapeDtypeStruct((2, WKR_PAD), jnp.float32),
            jax.ShapeDtypeStruct((2, DEGT_PAD), jnp.float32),
        ),
        mesh=mesh,
        scratch_types=(
            pltpu.VMEM((C2,), jnp.int32),
            pltpu.VMEM((C2,), jnp.int32),
            pltpu.VMEM((C2,), jnp.int32),
            pltpu.VMEM((128,), jnp.int32),
            pltpu.VMEM((128,), jnp.int32),
            pltpu.VMEM((128,), jnp.int32),
            pltpu.VMEM((128,), jnp.int32),
            pltpu.VMEM((128,), jnp.int32),
            pltpu.VMEM((128,), jnp.int32),
            pltpu.VMEM((128,), jnp.float32),
            pltpu.VMEM((128, 128), jnp.float32),
            pltpu.VMEM((128, 128), jnp.float32),
            pltpu.VMEM((128, 128), jnp.float32),
            pltpu.VMEM((128, 128), jnp.float32),
            pltpu.VMEM((64, 128), jnp.float32),
            pltpu.VMEM_SHARED((WKR_PAD, 128), jnp.float32),
            pltpu.VMEM_SHARED((WKR_PAD, 128), jnp.float32),
            pltpu.VMEM_SHARED((WKR_PAD,), jnp.float32),
            pltpu.VMEM_SHARED((DEGT_PAD,), jnp.float32),
            pltpu.SemaphoreType.DMA,
        ),
    )
    def k(t1_hbm, src_hbm, dst_hbm, typ_hbm,
          abil_out, degw_out, degt_out,
          srcv, dstv, typv, keyA, keyB, srcA, srcB, dstA, dstB, onesv,
          rowsAL, rowsAR, rowsBL, rowsBR, zbuf,
          accL, accR, degw_s, degt_s, sem):
        c = lax.axis_index("c")
        s = lax.axis_index("s")
        wid = c * 16 + s

        z16 = jnp.zeros((16,), jnp.float32)
        o16 = jnp.ones((16,), jnp.float32)

        def zrow(i, carry):
            for j in range(8):
                zbuf[i, pl.ds(j * 16, 16)] = z16
            return carry
        lax.fori_loop(0, 64, zrow, 0)
        for j in range(8):
            onesv[pl.ds(j * 16, 16)] = o16

        for half in (accL, accR):
            pltpu.sync_copy(zbuf, half.at[pl.ds(s * 128, 64)])
            pltpu.sync_copy(zbuf, half.at[pl.ds(s * 128 + 64, 64)])
        pltpu.sync_copy(zbuf.at[0], degw_s.at[pl.ds(s * 128, 128)])
        for b in range(4):
            pltpu.sync_copy(zbuf.at[b],
                            degt_s.at[pl.ds(s * 512 + b * 128, 128)])
        plsc.subcore_barrier()

        def chunk(j, carry):
            cid = wid + j * NTILES

            @pl.when(cid < NCHUNK2)
            def _():
                base = cid * C2
                ld1 = pltpu.async_copy(src_hbm.at[pl.ds(base, C2)], srcv, sem)
                ld2 = pltpu.async_copy(dst_hbm.at[pl.ds(base, C2)], dstv, sem)
                ld3 = pltpu.async_copy(typ_hbm.at[pl.ds(base, C2)], typv, sem)
                ld1.wait()
                ld2.wait()
                ld3.wait()
                for i in range(8):
                    sl = pl.ds(i * 16, 16)
                    sh = pl.ds(128 + i * 16, 16)
                    keyA[sl] = typv[sl] * NUM_TSK + srcv[sl]
                    keyB[sl] = typv[sh] * NUM_TSK + srcv[sh]
                    srcA[sl] = srcv[sl]
                    srcB[sl] = srcv[sh]
                    dstA[sl] = dstv[sl]
                    dstB[sl] = dstv[sh]
                g1 = pltpu.async_copy(
                    t1_hbm.at[keyA, pl.ds(0, 128)], rowsAL, sem)
                g2 = pltpu.async_copy(
                    t1_hbm.at[keyA, pl.ds(128, 128)], rowsAR, sem)
                g3 = pltpu.async_copy(
                    t1_hbm.at[keyB, pl.ds(0, 128)], rowsBL, sem)
                g4 = pltpu.async_copy(
                    t1_hbm.at[keyB, pl.ds(128, 128)], rowsBR, sem)
                g1.wait()
                g2.wait()
                g3.wait()
                g4.wait()
                s1 = pltpu.async_copy(rowsAL, accL.at[dstA], sem, add=True)
                s2 = pltpu.async_copy(rowsAR, accR.at[dstA], sem, add=True)
                s3 = pltpu.async_copy(rowsBL, accL.at[dstB], sem, add=True)
                s4 = pltpu.async_copy(rowsBR, accR.at[dstB], sem, add=True)
                pltpu.sync_copy(onesv, degw_s.at[dstA], add=True)
                pltpu.sync_copy(onesv, degw_s.at[dstB], add=True)
                pltpu.sync_copy(onesv, degt_s.at[srcA], add=True)
                pltpu.sync_copy(onesv, degt_s.at[srcB], add=True)
                s1.wait()
                s2.wait()
                s3.wait()
                s4.wait()
            return carry
        lax.fori_loop(0, -(-NCHUNK2 // NTILES), chunk, 0)

        plsc.subcore_barrier()
        pltpu.sync_copy(accL.at[pl.ds(s * 128, 128)],
                        abil_out.at[c, pl.ds(s * 128, 128), pl.ds(0, 128)])
        pltpu.sync_copy(accR.at[pl.ds(s * 128, 128)],
                        abil_out.at[c, pl.ds(s * 128, 128), pl.ds(128, 128)])
        pltpu.sync_copy(degw_s.at[pl.ds(s * 128, 128)],
                        degw_out.at[c, pl.ds(s * 128, 128)])
        pltpu.sync_copy(degt_s.at[pl.ds(s * 512, 512)],
                        degt_out.at[c, pl.ds(s * 512, 512)])

    return k(t1_flat, esrc, edst, etyp)


def _sc_pass2(t2_flat, esrc, edst, etyp):
    mesh = plsc.VectorSubcoreMesh(core_axis_name="c", subcore_axis_name="s")

    C2 = 256
    NCHUNK2 = E // C2

    @functools.partial(
        pl.kernel,
        out_type=jax.ShapeDtypeStruct((2, TSK_PAD2, 128), jnp.float32),
        mesh=mesh,
        scratch_types=(
            pltpu.VMEM((C2,), jnp.int32),
            pltpu.VMEM((C2,), jnp.int32),
            pltpu.VMEM((C2,), jnp.int32),
            pltpu.VMEM((128,), jnp.int32),
            pltpu.VMEM((128,), jnp.int32),
            pltpu.VMEM((128,), jnp.int32),
            pltpu.VMEM((128,), jnp.int32),
            pltpu.VMEM((128, 128), jnp.float32),
            pltpu.VMEM((128, 128), jnp.float32),
            pltpu.VMEM((64, 128), jnp.float32),
            pltpu.VMEM_SHARED((TSK_PAD2, 128), jnp.float32),
            pltpu.SemaphoreType.DMA,
        ),
    )
    def k(t2_hbm, src_hbm, dst_hbm, typ_hbm, out_hbm,
          srcv, dstv, typv, keyA, keyB, srcA, srcB,
          rowsA, rowsB, zbuf, acc, sem):
        c = lax.axis_index("c")
        s = lax.axis_index("s")
        colbase = pl.multiple_of(c * 128, 128)

        z16 = jnp.zeros((16,), jnp.float32)

        def zrow(i, carry):
            for j in range(8):
                zbuf[i, pl.ds(j * 16, 16)] = z16
            return carry
        lax.fori_loop(0, 64, zrow, 0)

        for b in range(8):
            pltpu.sync_copy(zbuf, acc.at[pl.ds(s * 512 + b * 64, 64)])
        plsc.subcore_barrier()

        def chunk(j, carry):
            cid = s + j * 16

            @pl.when(cid < NCHUNK2)
            def _():
                base = cid * C2
                ld1 = pltpu.async_copy(src_hbm.at[pl.ds(base, C2)], srcv, sem)
                ld2 = pltpu.async_copy(dst_hbm.at[pl.ds(base, C2)], dstv, sem)
                ld3 = pltpu.async_copy(typ_hbm.at[pl.ds(base, C2)], typv, sem)
                ld1.wait()
                ld2.wait()
                ld3.wait()
                for i in range(8):
                    sl = pl.ds(i * 16, 16)
                    sh = pl.ds(128 + i * 16, 16)
                    keyA[sl] = typv[sl] * NUM_WKR + dstv[sl]
                    keyB[sl] = typv[sh] * NUM_WKR + dstv[sh]
                    srcA[sl] = srcv[sl]
                    srcB[sl] = srcv[sh]
                cpA = pltpu.async_copy(
                    t2_hbm.at[keyA, pl.ds(colbase, 128)], rowsA, sem)
                cpB = pltpu.async_copy(
                    t2_hbm.at[keyB, pl.ds(colbase, 128)], rowsB, sem)
                cpA.wait()
                cpB.wait()
                scA = pltpu.async_copy(rowsA, acc.at[srcA], sem, add=True)
                scB = pltpu.async_copy(rowsB, acc.at[srcB], sem, add=True)
                scA.wait()
                scB.wait()
            return carry
        lax.fori_loop(0, -(-NCHUNK2 // 16), chunk, 0)

        plsc.subcore_barrier()
        pltpu.sync_copy(acc.at[pl.ds(s * 512, 512)],
                        out_hbm.at[c, pl.ds(s * 512, 512)])

    return k(t2_flat, esrc, edst, etyp)


def kernel(labels, edge_src_task, edge_dst_wkr, edge_type,
           weight_worker, weight_task):
    esrc = edge_src_task.astype(jnp.int32)
    edst = edge_dst_wkr.astype(jnp.int32)
    etyp = edge_type.astype(jnp.int32)

    t1 = _tc_transform1(labels, weight_worker)
    abil_p, degw_p, degt_p = _sc_pass1(
        t1.reshape(NUM_RELS * NUM_TSK, DIM), esrc, edst, etyp)

    ability, t2 = _tc_normalize_transform2(
        abil_p, degw_p.reshape(2, WKR_PAD, 1), weight_task)

    q_p = _sc_pass2(t2.reshape(NUM_RELS * NUM_WKR, DIM), esrc, edst, etyp)
    new_labels = _tc_normalize(q_p, degt_p.reshape(2, DEGT_PAD, 1))
    return ability, new_labels
